# Initial kernel scaffold; baseline (speedup 1.0000x reference)
import jax
import jax.numpy as jnp
from jax.experimental import pallas as pl


def kernel(x, batch, W1a, b1a, W1b, b1b, W2a, b2a, W2b, b2b, Wf, bf):
    raise NotImplementedError("write your pallas kernel here")


# speedup vs baseline: 8.9822x; 8.9822x over previous
import functools

import jax
import jax.numpy as jnp
from jax.experimental import pallas as pl
from jax.experimental.pallas import tpu as pltpu

_G = 256
_N = 128
_D = 128
_K = 16


def _edge_conv(xg, Wdiff, Wlo, ba, Wb, bb):
    f32 = jnp.float32
    sq = jnp.sum(xg * xg, axis=-1)
    prod = jax.lax.dot_general(
        xg, xg, (((1,), (1,)), ((), ())), preferred_element_type=f32)
    dist = sq[:, None] + sq[None, :] - 2.0 * prod
    row = jax.lax.broadcasted_iota(jnp.int32, (_N, _N), 0)
    col = jax.lax.broadcasted_iota(jnp.int32, (_N, _N), 1)
    dist = jnp.where(row == col, dist + 1e9, dist)

    P = jnp.dot(xg, Wdiff, preferred_element_type=f32) + ba[None, :]
    Q = jnp.dot(xg, Wlo, preferred_element_type=f32)

    def body(_, carry):
        dist, acc = carry
        m = jnp.min(dist, axis=-1, keepdims=True)
        idx = jnp.min(jnp.where(dist == m, col, _N), axis=-1, keepdims=True)
        sel = col == idx
        oh = sel.astype(f32)
        dist = jnp.where(sel, f32(jnp.inf), dist)
        qk = jnp.dot(oh, Q, preferred_element_type=f32)
        h = jnp.maximum(P + qk, 0.0)
        acc = acc + jnp.maximum(
            jnp.dot(h, Wb, preferred_element_type=f32) + bb[None, :], 0.0)
        return dist, acc

    _, acc = jax.lax.fori_loop(
        0, _K, body, (dist, jnp.zeros((_N, _D), f32)), unroll=True)
    return acc * (1.0 / _K)


def _gnn_kernel(x_ref, Wd1_ref, Wl1_ref, b1a_ref, Wb1_ref, b1b_ref,
                Wd2_ref, Wl2_ref, b2a_ref, Wb2_ref, b2b_ref,
                Wf_ref, bf_ref, out_ref):
    xg = x_ref[0]
    h1 = _edge_conv(xg, Wd1_ref[...], Wl1_ref[...], b1a_ref[...],
                    Wb1_ref[...], b1b_ref[...])
    h2 = _edge_conv(h1, Wd2_ref[...], Wl2_ref[...], b2a_ref[...],
                    Wb2_ref[...], b2b_ref[...])
    pooled = jnp.sum(h2, axis=0, keepdims=True) * (1.0 / _N)
    out_ref[0] = (
        jnp.dot(pooled, Wf_ref[...], preferred_element_type=jnp.float32)
        + bf_ref[...][None, :])


@jax.jit
def kernel(x, batch, W1a, b1a, W1b, b1b, W2a, b2a, W2b, b2b, Wf, bf):
    del batch
    xg = x.reshape(_G, _N, _D)
    Wd1, Wl1 = W1a[:_D] - W1a[_D:], W1a[_D:]
    Wd2, Wl2 = W2a[:_D] - W2a[_D:], W2a[_D:]

    wspec = pl.BlockSpec((_D, _D), lambda g: (0, 0))
    bspec = pl.BlockSpec((_D,), lambda g: (0,))
    out = pl.pallas_call(
        _gnn_kernel,
        grid=(_G,),
        in_specs=[
            pl.BlockSpec((1, _N, _D), lambda g: (g, 0, 0)),
            wspec, wspec, bspec, wspec, bspec,
            wspec, wspec, bspec, wspec, bspec,
            wspec, bspec,
        ],
        out_specs=pl.BlockSpec((1, 1, _D), lambda g: (g, 0, 0)),
        out_shape=jax.ShapeDtypeStruct((_G, 1, _D), jnp.float32),
    )(xg, Wd1, Wl1, b1a, W1b, b1b, Wd2, Wl2, b2a, W2b, b2b, Wf, bf)
    return out.reshape(_G, _D)


# speedup vs baseline: 49.3702x; 5.4965x over previous
import functools

import jax
import jax.numpy as jnp
from jax.experimental import pallas as pl
from jax.experimental.pallas import tpu as pltpu

_G = 256
_N = 128
_D = 128
_K = 16


def _edge_conv_multi(xgs, Wdiff, Wlo, ba, Wb, bb):
    f32 = jnp.float32
    col = jax.lax.broadcasted_iota(jnp.int32, (_N, _N), 1)
    row = jax.lax.broadcasted_iota(jnp.int32, (_N, _N), 0)
    colf = col.astype(f32)
    diag = jnp.where(row == col, f32(1e9), f32(0.0))

    dists, Ps, Qs = [], [], []
    for xg in xgs:
        sq = jnp.sum(xg * xg, axis=-1)
        prod = jax.lax.dot_general(
            xg, xg, (((1,), (1,)), ((), ())), preferred_element_type=f32)
        dists.append(sq[:, None] + sq[None, :] - 2.0 * prod + diag)
        Ps.append(jnp.dot(xg, Wdiff, preferred_element_type=f32) + ba[None, :])
        Qs.append(jnp.dot(xg, Wlo, preferred_element_type=f32))

    def body(_, carry):
        dists, accs = carry
        new_d, new_a = [], []
        for dist, acc, P, Q in zip(dists, accs, Ps, Qs):
            m = jnp.min(dist, axis=-1, keepdims=True)
            idx = jnp.min(jnp.where(dist == m, colf, f32(_N)), axis=-1,
                          keepdims=True)
            sel = colf == idx
            oh = sel.astype(f32)
            new_d.append(jnp.where(sel, f32(jnp.inf), dist))
            qk = jnp.dot(oh, Q, preferred_element_type=f32)
            h = jnp.maximum(P + qk, 0.0)
            new_a.append(acc + jnp.maximum(
                jnp.dot(h, Wb, preferred_element_type=f32) + bb[None, :], 0.0))
        return tuple(new_d), tuple(new_a)

    zero = jnp.zeros((_N, _D), f32)
    _, accs = jax.lax.fori_loop(
        0, _K, body, (tuple(dists), (zero,) * len(xgs)), unroll=True)
    return [acc * (1.0 / _K) for acc in accs]


_GPB = 4


def _gnn_kernel(x_ref, Wd1_ref, Wl1_ref, b1a_ref, Wb1_ref, b1b_ref,
                Wd2_ref, Wl2_ref, b2a_ref, Wb2_ref, b2b_ref,
                Wf_ref, bf_ref, out_ref):
    xgs = [x_ref[0, i * _N:(i + 1) * _N, :] for i in range(_GPB)]
    h1s = _edge_conv_multi(xgs, Wd1_ref[...], Wl1_ref[...], b1a_ref[...],
                           Wb1_ref[...], b1b_ref[...])
    h2s = _edge_conv_multi(h1s, Wd2_ref[...], Wl2_ref[...], b2a_ref[...],
                           Wb2_ref[...], b2b_ref[...])
    for i in range(_GPB):
        pooled = jnp.sum(h2s[i], axis=0, keepdims=True) * (1.0 / _N)
        out_ref[0, i] = (
            jnp.dot(pooled, Wf_ref[...], preferred_element_type=jnp.float32)
            + bf_ref[...][None, :])[0]


@jax.jit
def kernel(x, batch, W1a, b1a, W1b, b1b, W2a, b2a, W2b, b2b, Wf, bf):
    del batch
    xg = x.reshape(_G, _N, _D)
    Wd1, Wl1 = W1a[:_D] - W1a[_D:], W1a[_D:]
    Wd2, Wl2 = W2a[:_D] - W2a[_D:], W2a[_D:]

    wspec = pl.BlockSpec((_D, _D), lambda g: (0, 0))
    bspec = pl.BlockSpec((_D,), lambda g: (0,))
    xb = xg.reshape(_G // _GPB, _GPB * _N, _D)
    out = pl.pallas_call(
        _gnn_kernel,
        grid=(_G // _GPB,),
        in_specs=[
            pl.BlockSpec((1, _GPB * _N, _D), lambda g: (g, 0, 0)),
            wspec, wspec, bspec, wspec, bspec,
            wspec, wspec, bspec, wspec, bspec,
            wspec, bspec,
        ],
        out_specs=pl.BlockSpec((1, _GPB, _D), lambda g: (g, 0, 0)),
        out_shape=jax.ShapeDtypeStruct((_G // _GPB, _GPB, _D), jnp.float32),
    )(xb, Wd1, Wl1, b1a, W1b, b1b, Wd2, Wl2, b2a, W2b, b2b, Wf, bf)
    return out.reshape(_G, _D)
